# baseline (speedup 1.0000x reference)
import jax
import jax.numpy as jnp
from jax.experimental import pallas as pl
from jax.experimental.pallas import tpu as pltpu

_NBUF = 8


def _qsample_body(t_ref, sa_ref, som_ref, x_hbm, n_hbm, o_hbm,
                  xb, nb, ob, xsem, nsem, osem):
    nchunks = t_ref.shape[0]

    def in_copies(c, slot):
        cx = pltpu.make_async_copy(x_hbm.at[c], xb.at[slot], xsem.at[slot])
        cn = pltpu.make_async_copy(n_hbm.at[c], nb.at[slot], nsem.at[slot])
        return cx, cn

    def out_copy(c, slot):
        return pltpu.make_async_copy(ob.at[slot], o_hbm.at[c], osem.at[slot])

    for b in range(_NBUF):
        cx, cn = in_copies(b, b)
        cx.start()
        cn.start()

    for c in range(nchunks):
        slot = c % _NBUF
        cx, cn = in_copies(c, slot)
        cx.wait()
        cn.wait()
        if c >= _NBUF:
            out_copy(c - _NBUF, slot).wait()
        tt = t_ref[c]
        ob[slot] = sa_ref[tt] * xb[slot] + som_ref[tt] * nb[slot]
        out_copy(c, slot).start()
        nxt = c + _NBUF
        if nxt < nchunks:
            cx2, cn2 = in_copies(nxt, slot)
            cx2.start()
            cn2.start()

    for c in range(max(nchunks - _NBUF, 0), nchunks):
        out_copy(c, c % _NBUF).wait()


def kernel(x_start, t, noise, sqrt_alphas_cumprod, sqrt_one_minus_alphas_cumprod):
    b, ch, h, w = x_start.shape
    smem = pl.BlockSpec(memory_space=pltpu.SMEM)
    hbm = pl.BlockSpec(memory_space=pltpu.MemorySpace.HBM)
    buf = pltpu.VMEM((_NBUF, ch, h, w), jnp.float32)
    return pl.pallas_call(
        _qsample_body,
        in_specs=[smem, smem, smem, hbm, hbm],
        out_specs=hbm,
        out_shape=jax.ShapeDtypeStruct((b, ch, h, w), jnp.float32),
        scratch_shapes=[
            buf, buf, buf,
            pltpu.SemaphoreType.DMA((_NBUF,)),
            pltpu.SemaphoreType.DMA((_NBUF,)),
            pltpu.SemaphoreType.DMA((_NBUF,)),
        ],
    )(t.astype(jnp.int32), sqrt_alphas_cumprod, sqrt_one_minus_alphas_cumprod,
      x_start, noise)
